# Initial kernel scaffold; baseline (speedup 1.0000x reference)
import jax
import jax.numpy as jnp
from jax.experimental import pallas as pl


def kernel(node_feat, W0, b0, W1, b1, W2, b2, W3, b3, Wp0, bp0, Wp1, bp1, Wp2, bp2, p_vec, conv1_w, conv1_b, conv2_w, conv2_b, out_w, out_b, edge_index):
    raise NotImplementedError("write your pallas kernel here")


# speedup vs baseline: 28.8946x; 28.8946x over previous
import functools

import jax
import jax.numpy as jnp
from jax import lax
from jax.experimental import pallas as pl
from jax.experimental.pallas import tpu as pltpu
from jax.experimental.pallas import tpu_sc as plsc

_G = 20
_n = 500
_Eg = 16000
_D = 128
_NP = 512
_QR = 128
_NW = 32
_NCHUNK = _G * (_NP // _QR)
_NEG = -1e30


def _build_adj(edge_index):
    mesh = plsc.VectorSubcoreMesh(core_axis_name="c", subcore_axis_name="s",
                                  num_cores=2, num_subcores=16)

    slab_words = _NP * _NP
    gpc = _G // 2
    ept = _Eg // 16
    eptp = 1024
    opt = slab_words // 16

    @functools.partial(
        pl.kernel,
        out_type=jax.ShapeDtypeStruct((_G * _NP * _NP,), jnp.float32),
        mesh=mesh,
        scratch_types=[
            pltpu.VMEM((eptp,), jnp.int32),
            pltpu.VMEM((eptp,), jnp.int32),
            pltpu.VMEM((eptp // 128, 128), jnp.int32),
            pltpu.VMEM((128,), jnp.float32),
            pltpu.VMEM((opt,), jnp.float32),
            pltpu.VMEM((opt,), jnp.float32),
            pltpu.VMEM_SHARED((slab_words + 128,), jnp.float32),
        ],
    )
    def adj_kernel(edges_hbm, out_hbm, src_v, dst_v, idx_v, ones_v,
                   zeros_v, buf_v, slab):
        cid = lax.axis_index("c")
        sid = lax.axis_index("s")
        lane = lax.iota(jnp.int32, 16)
        for j in range(8):
            ones_v[pl.ds(j * 16, 16)] = jnp.ones((16,), jnp.float32)

        def zb(i, carry):
            for j in range(32):
                zeros_v[pl.ds((i * 32 + j) * 16, 16)] = \
                    jnp.zeros((16,), jnp.float32)
            return carry

        lax.fori_loop(0, opt // 512, zb, 0)

        def do_graph(gl, carry):
            g = cid * gpc + gl
            pltpu.sync_copy(zeros_v, slab.at[pl.ds(sid * opt, opt)])
            plsc.subcore_barrier()
            e0 = g * _Eg + sid * ept
            pltpu.sync_copy(edges_hbm.at[pl.ds(e0, ept)],
                            src_v.at[pl.ds(0, ept)])
            pltpu.sync_copy(edges_hbm.at[pl.ds(_G * _Eg + e0, ept)],
                            dst_v.at[pl.ds(0, ept)])
            base = g * _n
            for step in range(eptp // 16):
                p = step * 16
                s = src_v[pl.ds(p, 16)]
                d = dst_v[pl.ds(p, 16)]
                valid = (lane + p) < ept
                lin = jnp.where(valid, (d - base) * _NP + (s - base),
                                slab_words + lane)
                idx_v[step // 8, pl.ds((step % 8) * 16, 16)] = lin
            for r in range(eptp // 128):
                pltpu.sync_copy(ones_v, slab.at[idx_v.at[r]], add=True)
            plsc.subcore_barrier()
            pltpu.sync_copy(slab.at[pl.ds(sid * opt, opt)], buf_v)
            pltpu.sync_copy(buf_v,
                            out_hbm.at[pl.ds(g * slab_words + sid * opt, opt)])
            return carry

        lax.fori_loop(0, gpc, do_graph, 0)

    return adj_kernel(edge_index.reshape(-1)).reshape(_G, _NP, _NP)


def _tc_body(a_ref, h_ref,
             w0, b0, w1, b1, w2, b2, w3, b3,
             wp0, bp0, wp1, bp1, wp2, bp2,
             p_ref, w1c, b1c, w2u, b2c, wp3, ob, o_ref):
    f32 = jnp.float32
    a = a_ref[0]
    h = h_ref[0]
    deg = jnp.sum(a, axis=1, keepdims=True) + 1.0

    x = h
    for w_r, b_r, keep in ((w0, b0, False), (w1, b1, False), (w2, b2, False),
                           (w3, b3, True), (wp0, bp0, False),
                           (wp1, bp1, False), (wp2, bp2, False)):
        t = jnp.dot(h, w_r[...], preferred_element_type=f32)
        s = jnp.dot(a, t, preferred_element_type=f32)
        h = jnp.tanh((s + t + b_r[...]) / deg)
        if keep:
            x = h

    sc = jnp.transpose(
        jnp.dot(h, p_ref[...], preferred_element_type=f32))
    col = lax.broadcasted_iota(jnp.int32, (1, _NP), 1)
    sc = jnp.where(col < _n, sc, _NEG)
    oh_rows = []
    val_list = []
    for k in range(30):
        m = jnp.max(sc)
        hit = sc == m
        idx = jnp.min(jnp.where(hit, col, _NP))
        sel = col == idx
        oh_rows.append(jnp.where(sel, 1.0, 0.0))
        val_list.append(jnp.full((1, 1), m, f32))
        sc = jnp.where(sel, _NEG, sc)
    oh = jnp.concatenate(oh_rows + [jnp.zeros((2, _NP), f32)], axis=0)
    vals = jnp.concatenate(val_list + [jnp.zeros((2, 1), f32)], axis=0)

    xsel = jnp.dot(oh, x, preferred_element_type=f32)
    xs = xsel * jnp.tanh(vals)
    c1 = jnp.maximum(jnp.dot(xs, w1c[...], preferred_element_type=f32)
                     + b1c[...], 0.0)

    r16 = lax.broadcasted_iota(jnp.int32, (16, 32), 0)
    c32 = lax.broadcasted_iota(jnp.int32, (16, 32), 1)
    pe = (c32 == 2 * r16).astype(f32)
    po = (c32 == 2 * r16 + 1).astype(f32)
    p1 = jnp.maximum(jnp.dot(pe, c1, preferred_element_type=f32),
                     jnp.dot(po, c1, preferred_element_type=f32))

    r11 = lax.broadcasted_iota(jnp.int32, (16, 16), 0)
    c16 = lax.broadcasted_iota(jnp.int32, (16, 16), 1)
    u = jnp.concatenate(
        [jnp.dot((c16 == r11 + dt).astype(f32), p1,
                 preferred_element_type=f32) for dt in range(5)],
        axis=1)
    c2 = jnp.maximum(jnp.dot(u, w2u[...], preferred_element_type=f32)
                     + b2c[...], 0.0)

    acc = jnp.zeros((1, 128), f32)
    for t in range(11):
        acc = acc + jnp.dot(c2[t:t + 1, :], wp3[t],
                            preferred_element_type=f32)
    o_ref[0] = jnp.maximum(acc + ob[...], 0.0)


def _tc_call(adj, h0, tc_args):
    full = lambda s: pl.BlockSpec(s, lambda g: (0,) * len(s))
    in_specs = [
        pl.BlockSpec((1, _NP, _NP), lambda g: (g, 0, 0)),
        pl.BlockSpec((1, _NP, _D), lambda g: (g, 0, 0)),
    ] + [full(a.shape) for a in tc_args]
    return pl.pallas_call(
        _tc_body,
        grid=(_G,),
        in_specs=in_specs,
        out_specs=pl.BlockSpec((1, 1, 128), lambda g: (g, 0, 0)),
        out_shape=jax.ShapeDtypeStruct((_G, 1, 128), jnp.float32),
    )(adj, h0, *tc_args)


def kernel(node_feat, W0, b0, W1, b1, W2, b2, W3, b3,
           Wp0, bp0, Wp1, bp1, Wp2, bp2, p_vec,
           conv1_w, conv1_b, conv2_w, conv2_b, out_w, out_b, edge_index):
    adj = _build_adj(edge_index)
    h0 = jnp.pad(node_feat.reshape(_G, _n, _D),
                 ((0, 0), (0, _NP - _n), (0, 0)))
    rb = lambda b: b.reshape(1, -1)
    w1c = conv1_w.reshape(16, 96).T
    w2u = conv2_w.transpose(2, 1, 0).reshape(80, 32)
    wp3 = out_w.reshape(32, 11, 128).transpose(1, 0, 2)
    tc_args = (W0, rb(b0), W1, rb(b1), W2, rb(b2), W3, rb(b3),
               Wp0, rb(bp0), Wp1, rb(bp1), Wp2, rb(bp2),
               p_vec.reshape(-1, 1), w1c, rb(conv1_b), w2u, rb(conv2_b),
               wp3, rb(out_b))
    out = _tc_call(adj, h0, tc_args)
    return out.reshape(_G, 128)
